# Initial kernel scaffold; baseline (speedup 1.0000x reference)
import jax
import jax.numpy as jnp
from jax.experimental import pallas as pl


def kernel(input_ids, speaker_ids, char_table, spk_table, ln_gamma, ln_beta):
    raise NotImplementedError("write your pallas kernel here")


# speedup vs baseline: 2.1730x; 2.1730x over previous
import functools

import jax
import jax.numpy as jnp
import numpy as np
from jax import lax
from jax.experimental import pallas as pl
from jax.experimental.pallas import tpu as pltpu
from jax.experimental.pallas import tpu_sc as plsc

DIM = 128
EPS = 1e-12
NC = 2
NS = 16
NW = NC * NS
NREG = DIM // 16
CH0 = 104


def _splat_sum16(x):
    lane = lax.iota(jnp.int32, 16)
    for k in range(4):
        x = x + x.at[lane ^ (1 << k)].get(mode="promise_in_bounds")
    return x


def _rsqrt16(v):
    i = lax.bitcast_convert_type(v, jnp.int32)
    y = lax.bitcast_convert_type(jnp.int32(0x5F3759DF) - (i >> 1), jnp.float32)
    for _ in range(3):
        y = y * (1.5 - 0.5 * v * y * y)
    return y


@functools.lru_cache(maxsize=None)
def _make_sc_kernel(B, L):
    rows_per_w = B // NW
    ch1 = L - CH0
    mesh = plsc.VectorSubcoreMesh(core_axis_name="c", subcore_axis_name="s")

    @functools.partial(
        pl.kernel,
        mesh=mesh,
        out_type=jax.ShapeDtypeStruct((B * L, DIM), jnp.float32),
        scratch_types=[
            pltpu.VMEM((L,), jnp.int32),
            pltpu.VMEM((L, DIM), jnp.float32),
            pltpu.VMEM((rows_per_w,), jnp.int32),
            pltpu.VMEM((rows_per_w, DIM), jnp.float32),
            pltpu.VMEM((DIM,), jnp.float32),
            pltpu.VMEM((DIM,), jnp.float32),
            pltpu.SemaphoreType.DMA,
        ],
    )
    def k(ids, spk_ids, char_tab, spk_tab, gamma, beta, out,
          idx_v, rows_v, sidx_v, srows_v, g_v, b_v, sem):
        wid = lax.axis_index("s") * NC + lax.axis_index("c")
        base_b = wid * rows_per_w

        pltpu.sync_copy(gamma, g_v)
        pltpu.sync_copy(beta, b_v)
        pltpu.sync_copy(spk_ids.at[pl.ds(base_b, rows_per_w)], sidx_v)
        pltpu.async_copy(spk_tab.at[sidx_v], srows_v, sem).wait()

        gk = [g_v[pl.ds(16 * j, 16)] for j in range(NREG)]
        bk = [b_v[pl.ds(16 * j, 16)] for j in range(NREG)]

        def row_body(r, carry):
            b = base_b + r
            pltpu.sync_copy(ids.at[pl.ds(b * L, L)], idx_v)
            c0 = pltpu.async_copy(
                char_tab.at[idx_v.at[pl.ds(0, CH0)]], rows_v.at[pl.ds(0, CH0)], sem)
            c1 = pltpu.async_copy(
                char_tab.at[idx_v.at[pl.ds(CH0, ch1)]], rows_v.at[pl.ds(CH0, ch1)], sem)
            c0.wait()
            c1.wait()
            sk = [srows_v[r, pl.ds(16 * j, 16)] for j in range(NREG)]

            def tok_body(t, tc):
                v = [rows_v[t, pl.ds(16 * j, 16)] + sk[j] for j in range(NREG)]
                acc = v[0]
                acc2 = v[0] * v[0]
                for j in range(1, NREG):
                    acc = acc + v[j]
                    acc2 = acc2 + v[j] * v[j]
                mean_v = _splat_sum16(acc) * (1.0 / DIM)
                var_v = _splat_sum16(acc2) * (1.0 / DIM) - mean_v * mean_v
                r_v = _rsqrt16(var_v + EPS)
                for j in range(NREG):
                    rows_v[t, pl.ds(16 * j, 16)] = (v[j] - mean_v) * (r_v * gk[j]) + bk[j]
                return tc

            lax.fori_loop(0, L, tok_body, 0)
            pltpu.sync_copy(rows_v, out.at[pl.ds(b * L, L)])
            return carry

        lax.fori_loop(0, rows_per_w, row_body, 0)

    return k


def kernel(input_ids, speaker_ids, char_table, spk_table, ln_gamma, ln_beta):
    B, L = input_ids.shape
    k = _make_sc_kernel(B, L)
    out = k(input_ids.reshape(B * L), speaker_ids, char_table, spk_table,
            ln_gamma, ln_beta)
    return out.reshape(B, L, DIM)


# speedup vs baseline: 3.3206x; 1.5281x over previous
import functools

import jax
import jax.numpy as jnp
import numpy as np
from jax import lax
from jax.experimental import pallas as pl
from jax.experimental.pallas import tpu as pltpu
from jax.experimental.pallas import tpu_sc as plsc

DIM = 128
EPS = 1e-12
NC = 2
NS = 16
NW = NC * NS
NREG = DIM // 16
CH0 = 104


def _splat_sum16(x):
    lane = lax.iota(jnp.int32, 16)
    for k in range(4):
        x = x + x.at[lane ^ (1 << k)].get(mode="promise_in_bounds")
    return x


def _rsqrt16(v):
    i = lax.bitcast_convert_type(v, jnp.int32)
    y = lax.bitcast_convert_type(jnp.int32(0x5F3759DF) - (i >> 1), jnp.float32)
    h = 0.5 * v
    for _ in range(2):
        y = y * (1.5 - h * y * y)
    return y


def _tree_sum(vs):
    while len(vs) > 1:
        vs = [vs[i] + vs[i + 1] for i in range(0, len(vs), 2)]
    return vs[0]


@functools.lru_cache(maxsize=None)
def _make_sc_kernel(B, L):
    rows_per_w = B // NW
    ch1 = L - CH0
    mesh = plsc.VectorSubcoreMesh(core_axis_name="c", subcore_axis_name="s")

    @functools.partial(
        pl.kernel,
        mesh=mesh,
        out_type=jax.ShapeDtypeStruct((B * L, DIM), jnp.float32),
        scratch_types=[
            pltpu.VMEM((L,), jnp.int32),
            pltpu.VMEM((L, DIM), jnp.float32),
            pltpu.VMEM((rows_per_w,), jnp.int32),
            pltpu.VMEM((rows_per_w, DIM), jnp.float32),
            pltpu.VMEM((DIM,), jnp.float32),
            pltpu.VMEM((DIM,), jnp.float32),
            pltpu.SemaphoreType.DMA,
        ],
    )
    def k(ids, spk_ids, char_tab, spk_tab, gamma, beta, out,
          idx_v, rows_v, sidx_v, srows_v, g_v, b_v, sem):
        wid = lax.axis_index("s") * NC + lax.axis_index("c")
        base_b = wid * rows_per_w

        pltpu.sync_copy(gamma, g_v)
        pltpu.sync_copy(beta, b_v)
        pltpu.sync_copy(spk_ids.at[pl.ds(base_b, rows_per_w)], sidx_v)
        pltpu.async_copy(spk_tab.at[sidx_v], srows_v, sem).wait()

        gk = [g_v[pl.ds(16 * j, 16)] for j in range(NREG)]
        bk = [b_v[pl.ds(16 * j, 16)] for j in range(NREG)]

        def row_body(r, carry):
            b = base_b + r
            pltpu.sync_copy(ids.at[pl.ds(b * L, L)], idx_v)
            c0 = pltpu.async_copy(
                char_tab.at[idx_v.at[pl.ds(0, CH0)]], rows_v.at[pl.ds(0, CH0)], sem)
            c1 = pltpu.async_copy(
                char_tab.at[idx_v.at[pl.ds(CH0, ch1)]], rows_v.at[pl.ds(CH0, ch1)], sem)
            c0.wait()
            c1.wait()
            sk = [srows_v[r, pl.ds(16 * j, 16)] for j in range(NREG)]

            @plsc.parallel_loop(0, L, 1, unroll=4)
            def tok_body(t):
                v = [rows_v[t, pl.ds(16 * j, 16)] + sk[j] for j in range(NREG)]
                acc = _tree_sum(v)
                acc2 = _tree_sum([x * x for x in v])
                mean_v = _splat_sum16(acc) * (1.0 / DIM)
                var_v = _splat_sum16(acc2) * (1.0 / DIM) - mean_v * mean_v
                r_v = _rsqrt16(var_v + EPS)
                for j in range(NREG):
                    rows_v[t, pl.ds(16 * j, 16)] = (v[j] - mean_v) * (r_v * gk[j]) + bk[j]
            pltpu.sync_copy(rows_v, out.at[pl.ds(b * L, L)])
            return carry

        lax.fori_loop(0, rows_per_w, row_body, 0)

    return k


def kernel(input_ids, speaker_ids, char_table, spk_table, ln_gamma, ln_beta):
    B, L = input_ids.shape
    k = _make_sc_kernel(B, L)
    out = k(input_ids.reshape(B * L), speaker_ids, char_table, spk_table,
            ln_gamma, ln_beta)
    return out.reshape(B, L, DIM)


# speedup vs baseline: 5.5211x; 1.6627x over previous
import functools

import jax
import jax.numpy as jnp
from jax import lax
from jax.experimental import pallas as pl
from jax.experimental.pallas import tpu as pltpu
from jax.experimental.pallas import tpu_sc as plsc

DIM = 128
EPS = 1e-12
NC = 2
NS = 16
NW = NC * NS
NREG = DIM // 16
CH0 = 104


def _splat_sum16(x):
    lane = lax.iota(jnp.int32, 16)
    for k in range(4):
        x = x + x.at[lane ^ (1 << k)].get(mode="promise_in_bounds")
    return x


def _rsqrt16(v):
    i = lax.bitcast_convert_type(v, jnp.int32)
    y = lax.bitcast_convert_type(jnp.int32(0x5F3759DF) - (i >> 1), jnp.float32)
    h = 0.5 * v
    for _ in range(2):
        y = y * (1.5 - h * y * y)
    return y


def _tree_sum(vs):
    while len(vs) > 1:
        vs = [vs[i] + vs[i + 1] for i in range(0, len(vs), 2)]
    return vs[0]


@functools.lru_cache(maxsize=None)
def _make_sc_kernel(B, L):
    rows_per_w = B // NW
    ch1 = L - CH0
    mesh = plsc.VectorSubcoreMesh(core_axis_name="c", subcore_axis_name="s")

    @functools.partial(
        pl.kernel,
        mesh=mesh,
        out_type=jax.ShapeDtypeStruct((B * L, DIM), jnp.float32),
        scratch_types=[
            pltpu.VMEM((rows_per_w * L,), jnp.int32),
            pltpu.VMEM((L, DIM), jnp.float32),
            pltpu.VMEM((L, DIM), jnp.float32),
            pltpu.VMEM((L, DIM), jnp.float32),
            pltpu.VMEM((L, DIM), jnp.float32),
            pltpu.VMEM((rows_per_w,), jnp.int32),
            pltpu.VMEM((rows_per_w, DIM), jnp.float32),
            pltpu.VMEM((DIM,), jnp.float32),
            pltpu.VMEM((DIM,), jnp.float32),
            pltpu.SemaphoreType.DMA,
            pltpu.SemaphoreType.DMA,
            pltpu.SemaphoreType.DMA,
            pltpu.SemaphoreType.DMA,
        ],
    )
    def k(ids, spk_ids, char_tab, spk_tab, gamma, beta, out,
          ids_v, gbA, gbB, obA, obB, sidx_v, srows_v, g_v, b_v,
          gsA, gsB, ssA, ssB):
        wid = lax.axis_index("s") * NC + lax.axis_index("c")
        base_b = wid * rows_per_w

        pltpu.sync_copy(gamma, g_v)
        pltpu.sync_copy(beta, b_v)
        pltpu.sync_copy(spk_ids.at[pl.ds(base_b, rows_per_w)], sidx_v)
        pltpu.sync_copy(ids.at[pl.ds(base_b * L, rows_per_w * L)], ids_v)
        pltpu.async_copy(spk_tab.at[sidx_v], srows_v, gsA).wait()

        gk = [g_v[pl.ds(16 * j, 16)] for j in range(NREG)]
        bk = [b_v[pl.ds(16 * j, 16)] for j in range(NREG)]

        def gather_descs(r, gb):
            return (
                (char_tab.at[ids_v.at[pl.ds(r * L, CH0)]], gb.at[pl.ds(0, CH0)]),
                (char_tab.at[ids_v.at[pl.ds(r * L + CH0, ch1)]], gb.at[pl.ds(CH0, ch1)]),
            )

        def issue_gather(r, gb, sem):
            for src, dst in gather_descs(r, gb):
                pltpu.async_copy(src, dst, sem)

        def drain_gather(r, gb, sem):
            for src, dst in gather_descs(r, gb):
                pltpu.make_async_copy(src, dst, sem).wait()

        def normalize_row(r, gb, ob):
            sk = [srows_v[r, pl.ds(16 * j, 16)] for j in range(NREG)]

            @plsc.parallel_loop(0, L, 1, unroll=4)
            def tok_body(t):
                v = [gb[t, pl.ds(16 * j, 16)] + sk[j] for j in range(NREG)]
                acc = _tree_sum(v)
                acc2 = _tree_sum([x * x for x in v])
                mean_v = _splat_sum16(acc) * (1.0 / DIM)
                var_v = _splat_sum16(acc2) * (1.0 / DIM) - mean_v * mean_v
                r_v = _rsqrt16(var_v + EPS)
                for j in range(NREG):
                    ob[t, pl.ds(16 * j, 16)] = (v[j] - mean_v) * (r_v * gk[j]) + bk[j]

        issue_gather(0, gbA, gsA)
        issue_gather(1, gbB, gsB)

        def grp(g, carry):
            for p, gb, ob, gs, ss in ((0, gbA, obA, gsA, ssA),
                                      (1, gbB, obB, gsB, ssB)):
                r = 2 * g + p
                drain_gather(r, gb, gs)

                @pl.when(r >= 2)
                def _():
                    pltpu.make_async_copy(ob, out.at[pl.ds(0, L)], ss).wait()

                normalize_row(r, gb, ob)
                pltpu.async_copy(ob, out.at[pl.ds((base_b + r) * L, L)], ss)

                @pl.when(r + 2 < rows_per_w)
                def _():
                    issue_gather(r + 2, gb, gs)
            return carry

        lax.fori_loop(0, rows_per_w // 2, grp, 0)
        pltpu.make_async_copy(obA, out.at[pl.ds(0, L)], ssA).wait()
        pltpu.make_async_copy(obB, out.at[pl.ds(0, L)], ssB).wait()

    return k


def kernel(input_ids, speaker_ids, char_table, spk_table, ln_gamma, ln_beta):
    B, L = input_ids.shape
    k = _make_sc_kernel(B, L)
    out = k(input_ids.reshape(B * L), speaker_ids, char_table, spk_table,
            ln_gamma, ln_beta)
    return out.reshape(B, L, DIM)
